# baseline (speedup 1.0000x reference)
import jax
import jax.numpy as jnp
from jax import lax
from jax.experimental import pallas as pl
from jax.experimental.pallas import tpu as pltpu
from jax.experimental.pallas import tpu_sc as plsc

_MU = 3.5
_B = 16384
_K = 90
_D = 128
_NC = 2
_NS = 16
_NW = _NC * _NS
_BPW = _B // _NW
_CH = 128
_NCHUNK = _BPW // _CH
_L = 16
_HALF = _BPW // 2


def _sc_body(p_hbm, q_hbm, bu_hbm, bi_hbm, uidx_hbm, iidx_hbm, out_hbm,
             uidx_v, iidx_v, p_v, q_v, bu_v, bi_v, out_v, sem):
    c = lax.axis_index("c")
    s = lax.axis_index("s")
    wid = s * _NC + c

    pltpu.sync_copy(uidx_hbm.at[pl.ds(wid * _NCHUNK, _NCHUNK)], uidx_v)
    pltpu.sync_copy(iidx_hbm.at[pl.ds(wid * _NCHUNK, _NCHUNK)], iidx_v)

    bias_copies = []
    for j in range(_NCHUNK):
        dst = pl.ds(j * _CH, _CH)
        bias_copies.append(pltpu.async_copy(bu_hbm.at[uidx_v.at[j]], bu_v.at[dst], sem))
        bias_copies.append(pltpu.async_copy(bi_hbm.at[iidx_v.at[j]], bi_v.at[dst], sem))

    lane = lax.iota(jnp.int32, _L)
    tail_mask = lane >= 6
    zero = jnp.zeros((_L,), jnp.float32)

    for half in range(2):
        row_copies = []
        for j in range(_HALF // _CH):
            jj = half * (_HALF // _CH) + j
            dst = pl.ds(j * _CH, _CH)
            row_copies.append(pltpu.async_copy(p_hbm.at[uidx_v.at[jj]], p_v.at[dst], sem))
            row_copies.append(pltpu.async_copy(q_hbm.at[iidx_v.at[jj]], q_v.at[dst], sem))
        if half == 0:
            for cp in bias_copies:
                cp.wait()
        for cp in row_copies:
            cp.wait()

        def group(g, carry):
            res = zero
            for j2 in range(_L):
                r = g * _L + j2
                acc = p_v[r, pl.ds(0, _L)] * q_v[r, pl.ds(0, _L)]
                for cix in range(1, 5):
                    acc = acc + p_v[r, pl.ds(cix * _L, _L)] * q_v[r, pl.ds(cix * _L, _L)]
                tp = p_v[r, pl.ds(74, _L)] * q_v[r, pl.ds(74, _L)]
                acc = acc + jnp.where(tail_mask, tp, zero)
                dot = jnp.sum(acc)
                res = jnp.where(lane == j2, dot, res)
            base = half * _HALF + g * _L
            out_v[pl.ds(base, _L)] = (res + bu_v[pl.ds(base, _L)]
                                      + bi_v[pl.ds(base, _L)] + jnp.float32(_MU))
            return carry

        lax.fori_loop(0, _HALF // _L, group, 0)

    pltpu.sync_copy(out_v, out_hbm.at[pl.ds(wid * _BPW, _BPW)])


@jax.jit
def _run(P, Q, b_u, b_i, uidx2, iidx2):
    mesh = plsc.VectorSubcoreMesh(core_axis_name="c", subcore_axis_name="s")
    f = pl.kernel(
        _sc_body,
        out_type=jax.ShapeDtypeStruct((_B,), jnp.float32),
        mesh=mesh,
        compiler_params=pltpu.CompilerParams(needs_layout_passes=False),
        scratch_types=[
            pltpu.VMEM((_NCHUNK, _CH), jnp.int32),
            pltpu.VMEM((_NCHUNK, _CH), jnp.int32),
            pltpu.VMEM((_HALF, _D), jnp.float32),
            pltpu.VMEM((_HALF, _D), jnp.float32),
            pltpu.VMEM((_BPW,), jnp.float32),
            pltpu.VMEM((_BPW,), jnp.float32),
            pltpu.VMEM((_BPW,), jnp.float32),
            pltpu.SemaphoreType.DMA,
        ],
    )
    return f(P, Q, b_u, b_i, uidx2, iidx2)


def kernel(P, Q, b_u, b_i, user_idx, item_idx):
    Pp = jnp.pad(P, ((0, 0), (0, _D - _K)))
    Qp = jnp.pad(Q, ((0, 0), (0, _D - _K)))
    uidx2 = user_idx.astype(jnp.int32).reshape(_B // _CH, _CH)
    iidx2 = item_idx.astype(jnp.int32).reshape(_B // _CH, _CH)
    return _run(Pp, Qp, b_u.reshape(-1), b_i.reshape(-1), uidx2, iidx2)


# speedup vs baseline: 1.6838x; 1.6838x over previous
import jax
import jax.numpy as jnp
from jax import lax
from jax.experimental import pallas as pl
from jax.experimental.pallas import tpu as pltpu
from jax.experimental.pallas import tpu_sc as plsc

_MU = 3.5
_B = 16384
_K = 90
_D = 128
_NC = 2
_NS = 16
_NW = _NC * _NS
_BPW = _B // _NW
_CH = 128
_NCHUNK = _BPW // _CH
_L = 16
_HALF = _BPW // 2


def _sc_body(p_hbm, q_hbm, bu_hbm, bi_hbm, uidx_hbm, iidx_hbm, out_hbm,
             uidx_v, iidx_v, p_v, q_v, bu_v, bi_v, out_v, sem):
    c = lax.axis_index("c")
    s = lax.axis_index("s")
    wid = s * _NC + c

    pltpu.sync_copy(uidx_hbm.at[pl.ds(wid * _NCHUNK, _NCHUNK)], uidx_v)
    pltpu.sync_copy(iidx_hbm.at[pl.ds(wid * _NCHUNK, _NCHUNK)], iidx_v)

    bias_copies = []
    for j in range(_NCHUNK):
        dst = pl.ds(j * _CH, _CH)
        bias_copies.append(pltpu.async_copy(bu_hbm.at[uidx_v.at[j]], bu_v.at[dst], sem))
        bias_copies.append(pltpu.async_copy(bi_hbm.at[iidx_v.at[j]], bi_v.at[dst], sem))

    lane = lax.iota(jnp.int32, _L)
    tail_mask = lane >= 6
    zero = jnp.zeros((_L,), jnp.float32)

    for half in range(2):
        row_copies = []
        for j in range(_HALF // _CH):
            jj = half * (_HALF // _CH) + j
            dst = pl.ds(j * _CH, _CH)
            row_copies.append(pltpu.async_copy(p_hbm.at[uidx_v.at[jj]], p_v.at[dst], sem))
            row_copies.append(pltpu.async_copy(q_hbm.at[iidx_v.at[jj]], q_v.at[dst], sem))
        if half == 0:
            for cp in bias_copies:
                cp.wait()
        for cp in row_copies:
            cp.wait()

        def group(g, carry):
            res = zero
            for j2 in range(_L):
                r = g * _L + j2
                acc = p_v[r, pl.ds(0, _L)] * q_v[r, pl.ds(0, _L)]
                for cix in range(1, 5):
                    acc = acc + p_v[r, pl.ds(cix * _L, _L)] * q_v[r, pl.ds(cix * _L, _L)]
                tp = p_v[r, pl.ds(74, _L)] * q_v[r, pl.ds(74, _L)]
                acc = acc + jnp.where(tail_mask, tp, zero)
                dot = jnp.sum(acc)
                res = jnp.where(lane == j2, dot, res)
            base = half * _HALF + g * _L
            out_v[pl.ds(base, _L)] = (res + bu_v[pl.ds(base, _L)]
                                      + bi_v[pl.ds(base, _L)] + jnp.float32(_MU))
            return carry

        lax.fori_loop(0, _HALF // _L, group, 0)

    pltpu.sync_copy(out_v, out_hbm.at[pl.ds(wid * _BPW, _BPW)])


_PAD_ROWS = 2000


def _pad_body(x_ref, o_ref):
    o_ref[:, : _K] = x_ref[...]
    o_ref[:, _K:] = jnp.zeros((_PAD_ROWS, _D - _K), jnp.float32)


def _pad_table(x):
    n = x.shape[0]
    return pl.pallas_call(
        _pad_body,
        grid=(n // _PAD_ROWS,),
        in_specs=[pl.BlockSpec((_PAD_ROWS, _K), lambda i: (i, 0))],
        out_specs=pl.BlockSpec((_PAD_ROWS, _D), lambda i: (i, 0)),
        out_shape=jax.ShapeDtypeStruct((n, _D), jnp.float32),
    )(x)


@jax.jit
def _run(P, Q, b_u, b_i, uidx2, iidx2):
    mesh = plsc.VectorSubcoreMesh(core_axis_name="c", subcore_axis_name="s")
    f = pl.kernel(
        _sc_body,
        out_type=jax.ShapeDtypeStruct((_B,), jnp.float32),
        mesh=mesh,
        compiler_params=pltpu.CompilerParams(needs_layout_passes=False),
        scratch_types=[
            pltpu.VMEM((_NCHUNK, _CH), jnp.int32),
            pltpu.VMEM((_NCHUNK, _CH), jnp.int32),
            pltpu.VMEM((_HALF, _D), jnp.float32),
            pltpu.VMEM((_HALF, _D), jnp.float32),
            pltpu.VMEM((_BPW,), jnp.float32),
            pltpu.VMEM((_BPW,), jnp.float32),
            pltpu.VMEM((_BPW,), jnp.float32),
            pltpu.SemaphoreType.DMA,
        ],
    )
    return f(_pad_table(P), _pad_table(Q), b_u, b_i, uidx2, iidx2)


def kernel(P, Q, b_u, b_i, user_idx, item_idx):
    uidx2 = user_idx.astype(jnp.int32).reshape(_B // _CH, _CH)
    iidx2 = item_idx.astype(jnp.int32).reshape(_B // _CH, _CH)
    return _run(P, Q, b_u.reshape(-1), b_i.reshape(-1), uidx2, iidx2)


# speedup vs baseline: 2.0658x; 1.2268x over previous
import jax
import jax.numpy as jnp
from jax import lax
from jax.experimental import pallas as pl
from jax.experimental.pallas import tpu as pltpu
from jax.experimental.pallas import tpu_sc as plsc

_MU = 3.5
_B = 16384
_K = 90
_D = 128
_NC = 2
_NS = 16
_NW = _NC * _NS
_BPW = _B // _NW
_CH = 128
_NCHUNK = _BPW // _CH
_L = 16
_HALF = _BPW // 2


def _sc_body(p_hbm, q_hbm, bu_hbm, bi_hbm, uidx_hbm, iidx_hbm, out_hbm,
             uidx_v, iidx_v, p_v, q_v, bu_v, bi_v, out_v, sem):
    c = lax.axis_index("c")
    s = lax.axis_index("s")
    wid = s * _NC + c

    pltpu.sync_copy(uidx_hbm.at[pl.ds(wid * _NCHUNK, _NCHUNK)], uidx_v)
    pltpu.sync_copy(iidx_hbm.at[pl.ds(wid * _NCHUNK, _NCHUNK)], iidx_v)

    bias_copies = []
    for j in range(_NCHUNK):
        dst = pl.ds(j * _CH, _CH)
        bias_copies.append(pltpu.async_copy(bu_hbm.at[uidx_v.at[j]], bu_v.at[dst], sem))
        bias_copies.append(pltpu.async_copy(bi_hbm.at[iidx_v.at[j]], bi_v.at[dst], sem))

    lane = lax.iota(jnp.int32, _L)
    tail_mask = lane >= 6
    zero = jnp.zeros((_L,), jnp.float32)

    for half in range(2):
        row_copies = []
        for j in range(_HALF // _CH):
            jj = half * (_HALF // _CH) + j
            dst = pl.ds(j * _CH, _CH)
            row_copies.append(pltpu.async_copy(p_hbm.at[uidx_v.at[jj]], p_v.at[dst], sem))
            row_copies.append(pltpu.async_copy(q_hbm.at[iidx_v.at[jj]], q_v.at[dst], sem))
        if half == 0:
            for cp in bias_copies:
                cp.wait()
        for cp in row_copies:
            cp.wait()

        def group(g, carry):
            res = zero
            for j2 in range(_L):
                r = g * _L + j2
                acc = p_v[r, pl.ds(0, _L)] * q_v[r, pl.ds(0, _L)]
                for cix in range(1, 5):
                    acc = acc + p_v[r, pl.ds(cix * _L, _L)] * q_v[r, pl.ds(cix * _L, _L)]
                tp = p_v[r, pl.ds(74, _L)] * q_v[r, pl.ds(74, _L)]
                acc = acc + jnp.where(tail_mask, tp, zero)
                dot = jnp.sum(acc)
                res = jnp.where(lane == j2, dot, res)
            base = half * _HALF + g * _L
            out_v[pl.ds(base, _L)] = (res + bu_v[pl.ds(base, _L)]
                                      + bi_v[pl.ds(base, _L)] + jnp.float32(_MU))
            return carry

        lax.fori_loop(0, _HALF // _L, group, 0)

    pltpu.sync_copy(out_v, out_hbm.at[pl.ds(wid * _BPW, _BPW)])


_PAD_ROWS = 10000


def _pad_body(x_ref, y_ref, ox_ref, oy_ref):
    zero_pad = jnp.zeros((_PAD_ROWS, _D - _K), jnp.float32)
    ox_ref[:, : _K] = x_ref[...]
    ox_ref[:, _K:] = zero_pad
    oy_ref[:, : _K] = y_ref[...]
    oy_ref[:, _K:] = zero_pad


def _pad_tables(x, y):
    n = x.shape[0]
    spec_in = pl.BlockSpec((_PAD_ROWS, _K), lambda i: (i, 0))
    spec_out = pl.BlockSpec((_PAD_ROWS, _D), lambda i: (i, 0))
    return pl.pallas_call(
        _pad_body,
        grid=(n // _PAD_ROWS,),
        in_specs=[spec_in, spec_in],
        out_specs=[spec_out, spec_out],
        out_shape=[jax.ShapeDtypeStruct((n, _D), jnp.float32)] * 2,
    )(x, y)


@jax.jit
def _run(P, Q, b_u, b_i, uidx2, iidx2):
    mesh = plsc.VectorSubcoreMesh(core_axis_name="c", subcore_axis_name="s")
    f = pl.kernel(
        _sc_body,
        out_type=jax.ShapeDtypeStruct((_B,), jnp.float32),
        mesh=mesh,
        compiler_params=pltpu.CompilerParams(needs_layout_passes=False),
        scratch_types=[
            pltpu.VMEM((_NCHUNK, _CH), jnp.int32),
            pltpu.VMEM((_NCHUNK, _CH), jnp.int32),
            pltpu.VMEM((_HALF, _D), jnp.float32),
            pltpu.VMEM((_HALF, _D), jnp.float32),
            pltpu.VMEM((_BPW,), jnp.float32),
            pltpu.VMEM((_BPW,), jnp.float32),
            pltpu.VMEM((_BPW,), jnp.float32),
            pltpu.SemaphoreType.DMA,
        ],
    )
    Pp, Qp = _pad_tables(P, Q)
    return f(Pp, Qp, b_u, b_i, uidx2, iidx2)


def kernel(P, Q, b_u, b_i, user_idx, item_idx):
    uidx2 = user_idx.astype(jnp.int32).reshape(_B // _CH, _CH)
    iidx2 = item_idx.astype(jnp.int32).reshape(_B // _CH, _CH)
    return _run(P, Q, b_u.reshape(-1), b_i.reshape(-1), uidx2, iidx2)


# speedup vs baseline: 2.9636x; 1.4346x over previous
import jax
import jax.numpy as jnp
from jax import lax
from jax.experimental import pallas as pl
from jax.experimental.pallas import tpu as pltpu
from jax.experimental.pallas import tpu_sc as plsc

_MU = 3.5
_B = 16384
_K = 90
_D = 128
_NC = 2
_NS = 16
_NW = _NC * _NS
_BPW = _B // _NW
_CH = 128
_NCHUNK = _BPW // _CH
_L = 16
_NG = _BPW // _L


def _sc_body(p_hbm, q_hbm, bu_hbm, bi_hbm, uidx_hbm, iidx_hbm, out_hbm,
             uidx_v, iidx_v, p_v, q_v, bu_v, bi_v, out_v, bsem, sem_a, sem_b):
    c = lax.axis_index("c")
    s = lax.axis_index("s")
    wid = s * _NC + c

    pltpu.sync_copy(uidx_hbm.at[pl.ds(wid * _NCHUNK, _NCHUNK)], uidx_v)
    pltpu.sync_copy(iidx_hbm.at[pl.ds(wid * _NCHUNK, _NCHUNK)], iidx_v)

    bias_copies = []
    for j in range(_NCHUNK):
        dst = pl.ds(j * _CH, _CH)
        bias_copies.append(pltpu.async_copy(bu_hbm.at[uidx_v.at[j]], bu_v.at[dst], bsem))
        bias_copies.append(pltpu.async_copy(bi_hbm.at[iidx_v.at[j]], bi_v.at[dst], bsem))

    lane = lax.iota(jnp.int32, _L)
    tail_mask = lane >= 6
    zero = jnp.zeros((_L,), jnp.float32)

    def issue(g, slot, sem):
        j0 = g // 8
        off = (g % 8) * _L
        uvec = uidx_v[j0, pl.ds(off, _L)]
        ivec = iidx_v[j0, pl.ds(off, _L)]
        for j in range(_L):
            u = uvec[j]
            i = ivec[j]
            row = slot * _L + j
            pltpu.async_copy(p_hbm.at[pl.ds(u, 1), :],
                             p_v.at[pl.ds(row, 1), pl.ds(0, _K)], sem)
            pltpu.async_copy(q_hbm.at[pl.ds(i, 1), :],
                             q_v.at[pl.ds(row, 1), pl.ds(0, _K)], sem)

    def drain(slot, sem):
        base = slot * _L
        pltpu.make_async_copy(p_hbm.at[pl.ds(0, _L), :],
                              p_v.at[pl.ds(base, _L), pl.ds(0, _K)], sem).wait()
        pltpu.make_async_copy(q_hbm.at[pl.ds(0, _L), :],
                              q_v.at[pl.ds(base, _L), pl.ds(0, _K)], sem).wait()

    def compute(g, slot):
        res = zero
        for j in range(_L):
            r = slot * _L + j
            acc = p_v[r, pl.ds(0, _L)] * q_v[r, pl.ds(0, _L)]
            for cix in range(1, 5):
                acc = acc + p_v[r, pl.ds(cix * _L, _L)] * q_v[r, pl.ds(cix * _L, _L)]
            tp = p_v[r, pl.ds(74, _L)] * q_v[r, pl.ds(74, _L)]
            acc = acc + jnp.where(tail_mask, tp, zero)
            res = jnp.where(lane == j, jnp.sum(acc), res)
        base = g * _L
        out_v[pl.ds(base, _L)] = (res + bu_v[pl.ds(base, _L)]
                                  + bi_v[pl.ds(base, _L)] + jnp.float32(_MU))
        return res

    for cp in bias_copies:
        cp.wait()

    issue(jnp.int32(0), 0, sem_a)

    def pair(g2, carry):
        g_even = g2 * 2
        g_odd = g_even + 1
        issue(g_odd, 1, sem_b)
        drain(0, sem_a)
        compute(g_even, 0)

        @pl.when(g2 < _NG // 2 - 1)
        def _():
            issue(g_odd + 1, 0, sem_a)

        drain(1, sem_b)
        compute(g_odd, 1)
        return carry

    lax.fori_loop(0, _NG // 2, pair, 0)

    pltpu.sync_copy(out_v, out_hbm.at[pl.ds(wid * _BPW, _BPW)])


@jax.jit
def _run(P, Q, b_u, b_i, uidx2, iidx2):
    mesh = plsc.VectorSubcoreMesh(core_axis_name="c", subcore_axis_name="s")
    f = pl.kernel(
        _sc_body,
        out_type=jax.ShapeDtypeStruct((_B,), jnp.float32),
        mesh=mesh,
        compiler_params=pltpu.CompilerParams(needs_layout_passes=False),
        scratch_types=[
            pltpu.VMEM((_NCHUNK, _CH), jnp.int32),
            pltpu.VMEM((_NCHUNK, _CH), jnp.int32),
            pltpu.VMEM((2 * _L, _K), jnp.float32),
            pltpu.VMEM((2 * _L, _K), jnp.float32),
            pltpu.VMEM((_BPW,), jnp.float32),
            pltpu.VMEM((_BPW,), jnp.float32),
            pltpu.VMEM((_BPW,), jnp.float32),
            pltpu.SemaphoreType.DMA,
            pltpu.SemaphoreType.DMA,
            pltpu.SemaphoreType.DMA,
        ],
    )
    return f(P, Q, b_u, b_i, uidx2, iidx2)


def kernel(P, Q, b_u, b_i, user_idx, item_idx):
    uidx2 = user_idx.astype(jnp.int32).reshape(_B // _CH, _CH)
    iidx2 = item_idx.astype(jnp.int32).reshape(_B // _CH, _CH)
    return _run(P, Q, b_u.reshape(-1), b_i.reshape(-1), uidx2, iidx2)
